# Initial kernel scaffold; baseline (speedup 1.0000x reference)
import jax
import jax.numpy as jnp
from jax.experimental import pallas as pl


def kernel(x, edge_index, W_self, W_neigh, b1, Wc1, bc1, prelu_a, Wc2, bc2):
    raise NotImplementedError("write your pallas kernel here")


# speedup vs baseline: 6.9739x; 6.9739x over previous
import functools

import jax
import jax.numpy as jnp
from jax import lax
from jax.experimental import pallas as pl
from jax.experimental.pallas import tpu as pltpu
from jax.experimental.pallas import tpu_sc as plsc

N = 10000
E = 320000
D = 128

NC = 2
NS = 16
NW = NC * NS
EPW = E // NW
CH = 128
NFULL = EPW // CH
TAIL = EPW - NFULL * CH
ZR = 1000
NZ = N // ZR


def _sc_agg_body(x_hbm, src_hbm, dst_hbm, zrow_hbm, zdeg_hbm,
                 agg_out, deg_out,
                 src_v, dst_v, rows_v, src_t, dst_t, rows_t, deg_v,
                 agg_sh, sem):
    c = lax.axis_index("c")
    s = lax.axis_index("s")
    w = c * NS + s
    base = w * EPW

    @pl.when(s < NZ)
    def _zero():
        start = pl.multiple_of(s * ZR, 8)
        pltpu.sync_copy(zrow_hbm, agg_sh.at[pl.ds(start, ZR)])

    pltpu.sync_copy(zdeg_hbm, deg_v)
    plsc.subcore_barrier()

    ones = jnp.full((16,), 1.0, jnp.float32)

    def full_chunk(j, carry):
        off = base + j * CH
        pltpu.sync_copy(src_hbm.at[pl.ds(off, CH)], src_v)
        pltpu.sync_copy(dst_hbm.at[pl.ds(off, CH)], dst_v)
        pltpu.async_copy(x_hbm.at[src_v], rows_v, sem).wait()
        pltpu.sync_copy(rows_v, agg_sh.at[dst_v], add=True)
        for i in range(CH // 16):
            idx16 = dst_v[pl.ds(i * 16, 16)]
            plsc.addupdate_scatter(deg_v, [idx16], ones)
        return carry

    lax.fori_loop(0, NFULL, full_chunk, 0)

    off = base + NFULL * CH
    pltpu.sync_copy(src_hbm.at[pl.ds(off, TAIL)], src_t)
    pltpu.sync_copy(dst_hbm.at[pl.ds(off, TAIL)], dst_t)
    pltpu.async_copy(x_hbm.at[src_t], rows_t, sem).wait()
    pltpu.sync_copy(rows_t, agg_sh.at[dst_t], add=True)
    idx16 = dst_t[...]
    plsc.addupdate_scatter(deg_v, [idx16], ones)

    plsc.subcore_barrier()

    @pl.when(s < NZ)
    def _writeout():
        row0 = pl.multiple_of(s * ZR, 8)
        pltpu.sync_copy(agg_sh.at[pl.ds(row0, ZR)],
                        agg_out.at[c, pl.ds(row0, ZR)])

    pltpu.sync_copy(deg_v, deg_out.at[w, 0])


@jax.jit
def _sc_agg(x, src, dst, zrow, zdeg):
    mesh = plsc.VectorSubcoreMesh(core_axis_name="c", subcore_axis_name="s")
    f = pl.kernel(
        _sc_agg_body,
        out_type=(
            jax.ShapeDtypeStruct((NC, N, D), jnp.float32),
            jax.ShapeDtypeStruct((NW, 1, N), jnp.float32),
        ),
        mesh=mesh,
        compiler_params=pltpu.CompilerParams(needs_layout_passes=False),
        scratch_types=[
            pltpu.VMEM((CH,), jnp.int32),
            pltpu.VMEM((CH,), jnp.int32),
            pltpu.VMEM((CH, D), jnp.float32),
            pltpu.VMEM((TAIL,), jnp.int32),
            pltpu.VMEM((TAIL,), jnp.int32),
            pltpu.VMEM((TAIL, D), jnp.float32),
            pltpu.VMEM((N,), jnp.float32),
            pltpu.VMEM_SHARED((N, D), jnp.float32),
            pltpu.SemaphoreType.DMA,
        ],
    )
    return f(x, src, dst, zrow, zdeg)


BLK = 2048


def _tc_mlp_body(x_ref, aggp_ref, degp_ref, ws_ref, wn_ref, b1_ref,
                 wc1_ref, bc1_ref, a_ref, wc2_ref, bc2_ref, out_ref):
    hi = jax.lax.Precision.HIGHEST
    x = x_ref[...]
    agg = aggp_ref[0] + aggp_ref[1]
    deg = jnp.sum(degp_ref[...], axis=0)
    agg = agg / jnp.maximum(deg, 1.0)[:, None]
    h1 = jax.nn.relu(
        jnp.dot(x, ws_ref[...], precision=hi,
                preferred_element_type=jnp.float32) + b1_ref[0, :D])
    h2 = jax.nn.relu(
        jnp.dot(agg, wn_ref[...], precision=hi,
                preferred_element_type=jnp.float32) + b1_ref[0, D:])
    z = (jnp.dot(h1, wc1_ref[pl.ds(0, D)], precision=hi,
                 preferred_element_type=jnp.float32)
         + jnp.dot(h2, wc1_ref[pl.ds(D, D)], precision=hi,
                   preferred_element_type=jnp.float32)
         + bc1_ref[0])
    z = jnp.where(z >= 0, z, a_ref[0] * z)
    out_ref[...] = (jnp.dot(z, wc2_ref[...], precision=hi,
                            preferred_element_type=jnp.float32) + bc2_ref[0])


@jax.jit
def _tc_mlp(x, agg_p, deg_p, W_self, W_neigh, b1, Wc1, bc1, prelu_a, Wc2, bc2):
    grid = ((N + BLK - 1) // BLK,)
    return pl.pallas_call(
        _tc_mlp_body,
        grid=grid,
        in_specs=[
            pl.BlockSpec((BLK, D), lambda i: (i, 0)),
            pl.BlockSpec((NC, BLK, D), lambda i: (0, i, 0)),
            pl.BlockSpec((NW, BLK), lambda i: (0, i)),
            pl.BlockSpec((D, D), lambda i: (0, 0)),
            pl.BlockSpec((D, D), lambda i: (0, 0)),
            pl.BlockSpec((1, 2 * D), lambda i: (0, 0)),
            pl.BlockSpec((2 * D, D), lambda i: (0, 0)),
            pl.BlockSpec((1, D), lambda i: (0, 0)),
            pl.BlockSpec((1, D), lambda i: (0, 0)),
            pl.BlockSpec((D, 2), lambda i: (0, 0)),
            pl.BlockSpec((1, 2), lambda i: (0, 0)),
        ],
        out_specs=pl.BlockSpec((BLK, 2), lambda i: (i, 0)),
        out_shape=jax.ShapeDtypeStruct((N, 2), jnp.float32),
    )(x, agg_p, deg_p, W_self, W_neigh, b1, Wc1, bc1, prelu_a, Wc2, bc2)


def kernel(x, edge_index, W_self, W_neigh, b1, Wc1, bc1, prelu_a, Wc2, bc2):
    src = edge_index[0]
    dst = edge_index[1]
    zrow = jnp.zeros((ZR, D), jnp.float32)
    zdeg = jnp.zeros((N,), jnp.float32)
    agg_p, deg_3d = _sc_agg(x, src, dst, zrow, zdeg)
    deg_p = deg_3d.reshape(NW, N)
    out = _tc_mlp(x, agg_p, deg_p, W_self, W_neigh,
                  b1.reshape(1, 2 * D), Wc1, bc1.reshape(1, D),
                  prelu_a.reshape(1, D), Wc2, bc2.reshape(1, 2))
    return out.reshape(-1)


# speedup vs baseline: 9.9732x; 1.4301x over previous
import functools

import jax
import jax.numpy as jnp
from jax import lax
from jax.experimental import pallas as pl
from jax.experimental.pallas import tpu as pltpu
from jax.experimental.pallas import tpu_sc as plsc

N = 10000
E = 320000
D = 128

NC = 2
NS = 16
NW = NC * NS
EPW = E // NW
CH = 128
NFULL = EPW // CH
TAIL = EPW - NFULL * CH
ZR = 1000
NZ = N // ZR


NPAIR = NFULL // 2


def _sc_agg_body(x_hbm, src_hbm, dst_hbm, zrow_hbm, zdeg_hbm,
                 agg_out, deg_out,
                 src_c, dst_c, rows, dst_t, rows_t, deg_v,
                 agg_sh, sem_a, sem_b):
    c = lax.axis_index("c")
    s = lax.axis_index("s")
    w = c * NS + s
    base = w * EPW
    sems = (sem_a, sem_b)

    @pl.when(s < NZ)
    def _zero():
        start = pl.multiple_of(s * ZR, 8)
        pltpu.sync_copy(zrow_hbm, agg_sh.at[pl.ds(start, ZR)])

    pltpu.sync_copy(zdeg_hbm, deg_v)
    plsc.subcore_barrier()

    ones = jnp.full((16,), 1.0, jnp.float32)

    def gather_desc(j, b):
        return pltpu.make_async_copy(
            x_hbm.at[src_c.at[b]], rows.at[b], sems[b])

    def start_gather(j, b):
        pltpu.sync_copy(src_hbm.at[pl.ds(base + j * CH, CH)], src_c.at[b])
        gather_desc(j, b).start()

    def process(j, b):
        pltpu.sync_copy(dst_hbm.at[pl.ds(base + j * CH, CH)], dst_c.at[b])
        gather_desc(j, b).wait()
        pltpu.sync_copy(rows.at[b], agg_sh.at[dst_c.at[b]], add=True)
        for i in range(CH // 16):
            plsc.addupdate_scatter(deg_v, [dst_c[b, pl.ds(i * 16, 16)]], ones)

    start_gather(0, 0)

    def pair(i, carry):
        j0 = i * 2
        start_gather(j0 + 1, 1)
        process(j0, 0)

        @pl.when(i < NPAIR - 1)
        def _prefetch():
            start_gather(j0 + 2, 0)

        process(j0 + 1, 1)
        return carry

    lax.fori_loop(0, NPAIR, pair, 0)

    off = NFULL * CH
    pltpu.sync_copy(dst_hbm.at[pl.ds(base + off, TAIL)], dst_t)
    pltpu.sync_copy(src_hbm.at[pl.ds(base + off, TAIL)], src_c.at[0, pl.ds(0, TAIL)])
    pltpu.async_copy(x_hbm.at[src_c.at[0, pl.ds(0, TAIL)]], rows_t, sem_a).wait()
    pltpu.sync_copy(rows_t, agg_sh.at[dst_t], add=True)
    plsc.addupdate_scatter(deg_v, [dst_t[...]], ones)

    plsc.subcore_barrier()

    @pl.when(s < NZ)
    def _writeout():
        row0 = pl.multiple_of(s * ZR, 8)
        pltpu.sync_copy(agg_sh.at[pl.ds(row0, ZR)],
                        agg_out.at[c, pl.ds(row0, ZR)])

    pltpu.sync_copy(deg_v, deg_out.at[w, 0])


@jax.jit
def _sc_agg(x, src, dst, zrow, zdeg):
    mesh = plsc.VectorSubcoreMesh(core_axis_name="c", subcore_axis_name="s")
    f = pl.kernel(
        _sc_agg_body,
        out_type=(
            jax.ShapeDtypeStruct((NC, N, D), jnp.float32),
            jax.ShapeDtypeStruct((NW, 1, N), jnp.float32),
        ),
        mesh=mesh,
        compiler_params=pltpu.CompilerParams(needs_layout_passes=False),
        scratch_types=[
            pltpu.VMEM((2, CH), jnp.int32),
            pltpu.VMEM((2, CH), jnp.int32),
            pltpu.VMEM((2, CH, D), jnp.float32),
            pltpu.VMEM((TAIL,), jnp.int32),
            pltpu.VMEM((TAIL, D), jnp.float32),
            pltpu.VMEM((N,), jnp.float32),
            pltpu.VMEM_SHARED((N, D), jnp.float32),
            pltpu.SemaphoreType.DMA,
            pltpu.SemaphoreType.DMA,
        ],
    )
    return f(x, src, dst, zrow, zdeg)


BLK = 2048


def _tc_mlp_body(x_ref, aggp_ref, degp_ref, ws_ref, wn_ref, b1_ref,
                 wc1_ref, bc1_ref, a_ref, wc2_ref, bc2_ref, out_ref):
    hi = jax.lax.Precision.HIGHEST
    x = x_ref[...]
    agg = aggp_ref[0] + aggp_ref[1]
    deg = jnp.sum(degp_ref[...], axis=0)
    agg = agg / jnp.maximum(deg, 1.0)[:, None]
    h1 = jax.nn.relu(
        jnp.dot(x, ws_ref[...], precision=hi,
                preferred_element_type=jnp.float32) + b1_ref[0, :D])
    h2 = jax.nn.relu(
        jnp.dot(agg, wn_ref[...], precision=hi,
                preferred_element_type=jnp.float32) + b1_ref[0, D:])
    z = (jnp.dot(h1, wc1_ref[pl.ds(0, D)], precision=hi,
                 preferred_element_type=jnp.float32)
         + jnp.dot(h2, wc1_ref[pl.ds(D, D)], precision=hi,
                   preferred_element_type=jnp.float32)
         + bc1_ref[0])
    z = jnp.where(z >= 0, z, a_ref[0] * z)
    out_ref[...] = (jnp.dot(z, wc2_ref[...], precision=hi,
                            preferred_element_type=jnp.float32) + bc2_ref[0])


@jax.jit
def _tc_mlp(x, agg_p, deg_p, W_self, W_neigh, b1, Wc1, bc1, prelu_a, Wc2, bc2):
    grid = ((N + BLK - 1) // BLK,)
    return pl.pallas_call(
        _tc_mlp_body,
        grid=grid,
        in_specs=[
            pl.BlockSpec((BLK, D), lambda i: (i, 0)),
            pl.BlockSpec((NC, BLK, D), lambda i: (0, i, 0)),
            pl.BlockSpec((NW, BLK), lambda i: (0, i)),
            pl.BlockSpec((D, D), lambda i: (0, 0)),
            pl.BlockSpec((D, D), lambda i: (0, 0)),
            pl.BlockSpec((1, 2 * D), lambda i: (0, 0)),
            pl.BlockSpec((2 * D, D), lambda i: (0, 0)),
            pl.BlockSpec((1, D), lambda i: (0, 0)),
            pl.BlockSpec((1, D), lambda i: (0, 0)),
            pl.BlockSpec((D, 2), lambda i: (0, 0)),
            pl.BlockSpec((1, 2), lambda i: (0, 0)),
        ],
        out_specs=pl.BlockSpec((BLK, 2), lambda i: (i, 0)),
        out_shape=jax.ShapeDtypeStruct((N, 2), jnp.float32),
    )(x, agg_p, deg_p, W_self, W_neigh, b1, Wc1, bc1, prelu_a, Wc2, bc2)


def kernel(x, edge_index, W_self, W_neigh, b1, Wc1, bc1, prelu_a, Wc2, bc2):
    src = edge_index[0]
    dst = edge_index[1]
    zrow = jnp.zeros((ZR, D), jnp.float32)
    zdeg = jnp.zeros((N,), jnp.float32)
    agg_p, deg_3d = _sc_agg(x, src, dst, zrow, zdeg)
    deg_p = deg_3d.reshape(NW, N)
    out = _tc_mlp(x, agg_p, deg_p, W_self, W_neigh,
                  b1.reshape(1, 2 * D), Wc1, bc1.reshape(1, D),
                  prelu_a.reshape(1, D), Wc2, bc2.reshape(1, 2))
    return out.reshape(-1)


# speedup vs baseline: 10.3401x; 1.0368x over previous
import functools

import jax
import jax.numpy as jnp
from jax import lax
from jax.experimental import pallas as pl
from jax.experimental.pallas import tpu as pltpu
from jax.experimental.pallas import tpu_sc as plsc

N = 10000
E = 320000
D = 128

NC = 2
NS = 16
NW = NC * NS
EPW = E // NW
CH = 80
NCH = EPW // CH
NB = 3
NTRI = (NCH - 2) // NB
ZR = 1000
NZ = N // ZR


def _sc_agg_body(x_hbm, pairs_hbm, zrow_hbm, zdeg_hbm,
                 agg_out, deg_out,
                 pairs_c, rows, deg_v, agg_sh,
                 sg0, sg1, sg2, ss0, ss1, ss2):
    c = lax.axis_index("c")
    s = lax.axis_index("s")
    w = c * NS + s
    sg = (sg0, sg1, sg2)
    ss = (ss0, ss1, ss2)

    @pl.when(s < NZ)
    def _zero():
        start = pl.multiple_of(s * ZR, 8)
        pltpu.sync_copy(zrow_hbm, agg_sh.at[pl.ds(start, ZR)])

    pltpu.sync_copy(zdeg_hbm, deg_v)
    plsc.subcore_barrier()

    ones = jnp.full((16,), 1.0, jnp.float32)

    def gather_desc(b):
        return pltpu.make_async_copy(
            x_hbm.at[pairs_c.at[b, 0]], rows.at[b], sg[b])

    def scatter_desc(b):
        return pltpu.make_async_copy(
            rows.at[b], agg_sh.at[pairs_c.at[b, 1]], ss[b])

    def load_and_gather(j, b):
        pltpu.sync_copy(pairs_hbm.at[w, j], pairs_c.at[b])
        gather_desc(b).start()

    def step(j, b, guard_drain=False, prefetch=True):
        gather_desc(b).wait()
        pltpu.async_copy(rows.at[b], agg_sh.at[pairs_c.at[b, 1]], ss[b],
                         add=True)
        for i in range(CH // 16):
            plsc.addupdate_scatter(
                deg_v, [pairs_c[b, 1, pl.ds(i * 16, 16)]], ones)
        bn = (b + 2) % NB
        if guard_drain:
            @pl.when(j >= 1)
            def _drain():
                scatter_desc(bn).wait()
        else:
            scatter_desc(bn).wait()
        if prefetch:
            load_and_gather(j + 2, bn)

    load_and_gather(0, 0)
    load_and_gather(1, 1)

    def triple(i, carry):
        j0 = i * NB
        step(j0, 0, guard_drain=True)
        step(j0 + 1, 1)
        step(j0 + 2, 2)
        return carry

    lax.fori_loop(0, NTRI, triple, 0)
    step(NCH - 2, (NCH - 2) % NB, prefetch=False)
    step(NCH - 1, (NCH - 1) % NB, prefetch=False)
    scatter_desc((NCH - 1) % NB).wait()

    plsc.subcore_barrier()

    @pl.when(s < NZ)
    def _writeout():
        row0 = pl.multiple_of(s * ZR, 8)
        pltpu.sync_copy(agg_sh.at[pl.ds(row0, ZR)],
                        agg_out.at[c, pl.ds(row0, ZR)])

    pltpu.sync_copy(deg_v, deg_out.at[w, 0])


@jax.jit
def _sc_agg(x, pairs, zrow, zdeg):
    mesh = plsc.VectorSubcoreMesh(core_axis_name="c", subcore_axis_name="s")
    f = pl.kernel(
        _sc_agg_body,
        out_type=(
            jax.ShapeDtypeStruct((NC, N, D), jnp.float32),
            jax.ShapeDtypeStruct((NW, 1, N), jnp.float32),
        ),
        mesh=mesh,
        compiler_params=pltpu.CompilerParams(needs_layout_passes=False),
        scratch_types=[
            pltpu.VMEM((NB, 2, CH), jnp.int32),
            pltpu.VMEM((NB, CH, D), jnp.float32),
            pltpu.VMEM((N,), jnp.float32),
            pltpu.VMEM_SHARED((N, D), jnp.float32),
            pltpu.SemaphoreType.DMA,
            pltpu.SemaphoreType.DMA,
            pltpu.SemaphoreType.DMA,
            pltpu.SemaphoreType.DMA,
            pltpu.SemaphoreType.DMA,
            pltpu.SemaphoreType.DMA,
        ],
    )
    return f(x, pairs, zrow, zdeg)


BLK = 2048


def _tc_mlp_body(x_ref, aggp_ref, degp_ref, ws_ref, wn_ref, b1_ref,
                 wc1_ref, bc1_ref, a_ref, wc2_ref, bc2_ref, out_ref):
    hi = jax.lax.Precision.HIGHEST
    x = x_ref[...]
    agg = aggp_ref[0] + aggp_ref[1]
    deg = jnp.sum(degp_ref[...], axis=0)
    agg = agg / jnp.maximum(deg, 1.0)[:, None]
    h1 = jax.nn.relu(
        jnp.dot(x, ws_ref[...], precision=hi,
                preferred_element_type=jnp.float32) + b1_ref[0, :D])
    h2 = jax.nn.relu(
        jnp.dot(agg, wn_ref[...], precision=hi,
                preferred_element_type=jnp.float32) + b1_ref[0, D:])
    z = (jnp.dot(h1, wc1_ref[pl.ds(0, D)], precision=hi,
                 preferred_element_type=jnp.float32)
         + jnp.dot(h2, wc1_ref[pl.ds(D, D)], precision=hi,
                   preferred_element_type=jnp.float32)
         + bc1_ref[0])
    z = jnp.where(z >= 0, z, a_ref[0] * z)
    out_ref[...] = (jnp.dot(z, wc2_ref[...], precision=hi,
                            preferred_element_type=jnp.float32) + bc2_ref[0])


@jax.jit
def _tc_mlp(x, agg_p, deg_p, W_self, W_neigh, b1, Wc1, bc1, prelu_a, Wc2, bc2):
    grid = ((N + BLK - 1) // BLK,)
    return pl.pallas_call(
        _tc_mlp_body,
        grid=grid,
        in_specs=[
            pl.BlockSpec((BLK, D), lambda i: (i, 0)),
            pl.BlockSpec((NC, BLK, D), lambda i: (0, i, 0)),
            pl.BlockSpec((NW, BLK), lambda i: (0, i)),
            pl.BlockSpec((D, D), lambda i: (0, 0)),
            pl.BlockSpec((D, D), lambda i: (0, 0)),
            pl.BlockSpec((1, 2 * D), lambda i: (0, 0)),
            pl.BlockSpec((2 * D, D), lambda i: (0, 0)),
            pl.BlockSpec((1, D), lambda i: (0, 0)),
            pl.BlockSpec((1, D), lambda i: (0, 0)),
            pl.BlockSpec((D, 2), lambda i: (0, 0)),
            pl.BlockSpec((1, 2), lambda i: (0, 0)),
        ],
        out_specs=pl.BlockSpec((BLK, 2), lambda i: (i, 0)),
        out_shape=jax.ShapeDtypeStruct((N, 2), jnp.float32),
    )(x, agg_p, deg_p, W_self, W_neigh, b1, Wc1, bc1, prelu_a, Wc2, bc2)


def kernel(x, edge_index, W_self, W_neigh, b1, Wc1, bc1, prelu_a, Wc2, bc2):
    pairs = jnp.stack(
        [edge_index[0].reshape(NW, NCH, CH),
         edge_index[1].reshape(NW, NCH, CH)], axis=2)
    zrow = jnp.zeros((ZR, D), jnp.float32)
    zdeg = jnp.zeros((N,), jnp.float32)
    agg_p, deg_3d = _sc_agg(x, pairs, zrow, zdeg)
    deg_p = deg_3d.reshape(NW, N)
    out = _tc_mlp(x, agg_p, deg_p, W_self, W_neigh,
                  b1.reshape(1, 2 * D), Wc1, bc1.reshape(1, D),
                  prelu_a.reshape(1, D), Wc2, bc2.reshape(1, 2))
    return out.reshape(-1)


# speedup vs baseline: 11.8095x; 1.1421x over previous
import functools

import jax
import jax.numpy as jnp
from jax import lax
from jax.experimental import pallas as pl
from jax.experimental.pallas import tpu as pltpu
from jax.experimental.pallas import tpu_sc as plsc

N = 10000
E = 320000
D = 128

NC = 2
NS = 16
NW = NC * NS
EPW = E // NW
CH = 80
NCH = EPW // CH
NB = 3
NTRI = (NCH - 2) // NB
ZR = 1000
NZ = N // ZR


def _sc_agg_body(x_hbm, pairs_hbm, zrow_hbm, zdeg_hbm,
                 agg_out, deg_out,
                 pairs_c, rows, deg_v, agg_sh,
                 sg0, sg1, sg2, ss0, ss1, ss2):
    c = lax.axis_index("c")
    s = lax.axis_index("s")
    w = c * NS + s
    sg = (sg0, sg1, sg2)
    ss = (ss0, ss1, ss2)

    @pl.when(s < NZ)
    def _zero():
        start = pl.multiple_of(s * ZR, 8)
        pltpu.sync_copy(zrow_hbm, agg_sh.at[pl.ds(start, ZR)])

    pltpu.sync_copy(zdeg_hbm, deg_v)
    plsc.subcore_barrier()

    ones = jnp.full((16,), 1.0, jnp.float32)

    def gather_desc(b):
        return pltpu.make_async_copy(
            x_hbm.at[pairs_c.at[b, 0]], rows.at[b], sg[b])

    def scatter_desc(b):
        return pltpu.make_async_copy(
            rows.at[b], agg_sh.at[pairs_c.at[b, 1]], ss[b])

    def load_and_gather(j, b):
        pltpu.sync_copy(pairs_hbm.at[w, j], pairs_c.at[b])
        gather_desc(b).start()

    def step(j, b, guard_drain=False, prefetch=True):
        gather_desc(b).wait()
        pltpu.async_copy(rows.at[b], agg_sh.at[pairs_c.at[b, 1]], ss[b],
                         add=True)
        for i in range(CH // 16):
            plsc.addupdate_scatter(
                deg_v, [pairs_c[b, 1, pl.ds(i * 16, 16)]], ones)
        bn = (b + 2) % NB
        if guard_drain:
            @pl.when(j >= 1)
            def _drain():
                scatter_desc(bn).wait()
        else:
            scatter_desc(bn).wait()
        if prefetch:
            load_and_gather(j + 2, bn)

    load_and_gather(0, 0)
    load_and_gather(1, 1)

    def triple(i, carry):
        j0 = i * NB
        step(j0, 0, guard_drain=True)
        step(j0 + 1, 1)
        step(j0 + 2, 2)
        return carry

    lax.fori_loop(0, NTRI, triple, 0)
    step(NCH - 2, (NCH - 2) % NB, prefetch=False)
    step(NCH - 1, (NCH - 1) % NB, prefetch=False)
    scatter_desc((NCH - 1) % NB).wait()

    plsc.subcore_barrier()

    @pl.when(s < NZ)
    def _writeout():
        row0 = pl.multiple_of(s * ZR, 8)
        pltpu.sync_copy(agg_sh.at[pl.ds(row0, ZR)],
                        agg_out.at[c, pl.ds(row0, ZR)])

    pltpu.sync_copy(deg_v, deg_out.at[w, 0])


@jax.jit
def _sc_agg(x, pairs, zrow, zdeg):
    mesh = plsc.VectorSubcoreMesh(core_axis_name="c", subcore_axis_name="s")
    f = pl.kernel(
        _sc_agg_body,
        out_type=(
            jax.ShapeDtypeStruct((NC, N, D), jnp.float32),
            jax.ShapeDtypeStruct((NW, 1, N), jnp.float32),
        ),
        mesh=mesh,
        compiler_params=pltpu.CompilerParams(needs_layout_passes=False),
        scratch_types=[
            pltpu.VMEM((NB, 2, CH), jnp.int32),
            pltpu.VMEM((NB, CH, D), jnp.float32),
            pltpu.VMEM((N,), jnp.float32),
            pltpu.VMEM_SHARED((N, D), jnp.float32),
            pltpu.SemaphoreType.DMA,
            pltpu.SemaphoreType.DMA,
            pltpu.SemaphoreType.DMA,
            pltpu.SemaphoreType.DMA,
            pltpu.SemaphoreType.DMA,
            pltpu.SemaphoreType.DMA,
        ],
    )
    return f(x, pairs, zrow, zdeg)


BLK = 2048


def _tc_mlp_body(x_ref, aggp_ref, degp_ref, ws_ref, wn_ref, b1_ref,
                 wc1_ref, bc1_ref, a_ref, wc2_ref, bc2_ref, out_ref):
    hi = jax.lax.Precision.HIGHEST
    x = x_ref[...]
    agg = aggp_ref[0] + aggp_ref[1]
    deg = jnp.sum(degp_ref[...], axis=0)
    agg = agg / jnp.maximum(deg, 1.0)[:, None]
    h1 = jax.nn.relu(
        jnp.dot(x, ws_ref[...], precision=jax.lax.Precision.DEFAULT,
                preferred_element_type=jnp.float32) + b1_ref[0, :D])
    h2 = jax.nn.relu(
        jnp.dot(agg, wn_ref[...], precision=jax.lax.Precision.DEFAULT,
                preferred_element_type=jnp.float32) + b1_ref[0, D:])
    z = (jnp.dot(h1, wc1_ref[pl.ds(0, D)], precision=jax.lax.Precision.DEFAULT,
                 preferred_element_type=jnp.float32)
         + jnp.dot(h2, wc1_ref[pl.ds(D, D)], precision=jax.lax.Precision.DEFAULT,
                   preferred_element_type=jnp.float32)
         + bc1_ref[0])
    z = jnp.where(z >= 0, z, a_ref[0] * z)
    out_ref[...] = (jnp.dot(z, wc2_ref[...], precision=jax.lax.Precision.DEFAULT,
                            preferred_element_type=jnp.float32) + bc2_ref[0])


@jax.jit
def _tc_mlp(x, agg_p, deg_p, W_self, W_neigh, b1, Wc1, bc1, prelu_a, Wc2, bc2):
    grid = ((N + BLK - 1) // BLK,)
    return pl.pallas_call(
        _tc_mlp_body,
        grid=grid,
        in_specs=[
            pl.BlockSpec((BLK, D), lambda i: (i, 0)),
            pl.BlockSpec((NC, BLK, D), lambda i: (0, i, 0)),
            pl.BlockSpec((NW, BLK), lambda i: (0, i)),
            pl.BlockSpec((D, D), lambda i: (0, 0)),
            pl.BlockSpec((D, D), lambda i: (0, 0)),
            pl.BlockSpec((1, 2 * D), lambda i: (0, 0)),
            pl.BlockSpec((2 * D, D), lambda i: (0, 0)),
            pl.BlockSpec((1, D), lambda i: (0, 0)),
            pl.BlockSpec((1, D), lambda i: (0, 0)),
            pl.BlockSpec((D, 2), lambda i: (0, 0)),
            pl.BlockSpec((1, 2), lambda i: (0, 0)),
        ],
        out_specs=pl.BlockSpec((BLK, 2), lambda i: (i, 0)),
        out_shape=jax.ShapeDtypeStruct((N, 2), jnp.float32),
    )(x, agg_p, deg_p, W_self, W_neigh, b1, Wc1, bc1, prelu_a, Wc2, bc2)


def kernel(x, edge_index, W_self, W_neigh, b1, Wc1, bc1, prelu_a, Wc2, bc2):
    pairs = jnp.stack(
        [edge_index[0].reshape(NW, NCH, CH),
         edge_index[1].reshape(NW, NCH, CH)], axis=2)
    zrow = jnp.zeros((ZR, D), jnp.float32)
    zdeg = jnp.zeros((N,), jnp.float32)
    agg_p, deg_3d = _sc_agg(x, pairs, zrow, zdeg)
    deg_p = deg_3d.reshape(NW, N)
    out = _tc_mlp(x, agg_p, deg_p, W_self, W_neigh,
                  b1.reshape(1, 2 * D), Wc1, bc1.reshape(1, D),
                  prelu_a.reshape(1, D), Wc2, bc2.reshape(1, 2))
    return out.reshape(-1)


# speedup vs baseline: 13.5861x; 1.1504x over previous
import functools

import jax
import jax.numpy as jnp
from jax import lax
from jax.experimental import pallas as pl
from jax.experimental.pallas import tpu as pltpu
from jax.experimental.pallas import tpu_sc as plsc

N = 10000
E = 320000
D = 128

NC = 2
NS = 16
NW = NC * NS
EPW = E // NW
CH = 80
NCH = EPW // CH
NB = 3
NP = 6
UNROLL = 6
NLOOP = 20
ZR = 1000
NZ = N // ZR


def _sc_agg_body(x_hbm, pairs_hbm, zrow_hbm, zdeg_hbm,
                 agg_out, deg_out,
                 pairs_c, rows, deg_v, agg_sh,
                 sg0, sg1, sg2, ss0, ss1, ss2,
                 sp0, sp1, sp2, sp3, sp4, sp5):
    c = lax.axis_index("c")
    s = lax.axis_index("s")
    w = c * NS + s
    sg = (sg0, sg1, sg2)
    ss = (ss0, ss1, ss2)
    sp = (sp0, sp1, sp2, sp3, sp4, sp5)

    @pl.when(s < NZ)
    def _zero():
        start = pl.multiple_of(s * ZR, 8)
        pltpu.sync_copy(zrow_hbm, agg_sh.at[pl.ds(start, ZR)])

    pltpu.sync_copy(zdeg_hbm, deg_v)
    plsc.subcore_barrier()

    ones = jnp.full((16,), 1.0, jnp.float32)

    def pairs_desc(j, p):
        return pltpu.make_async_copy(pairs_hbm.at[w, j], pairs_c.at[p], sp[p])

    def gather_desc(b, p):
        return pltpu.make_async_copy(
            x_hbm.at[pairs_c.at[p, 0]], rows.at[b], sg[b])

    def scatter_desc(b, p):
        return pltpu.make_async_copy(
            rows.at[b], agg_sh.at[pairs_c.at[p, 1]], ss[b])

    def step(j, b, p, guard_drain=False, start2=True, load4=True):
        gather_desc(b, p).wait()
        pltpu.async_copy(rows.at[b], agg_sh.at[pairs_c.at[p, 1]], ss[b],
                         add=True)
        for i in range(CH // 16):
            plsc.addupdate_scatter(
                deg_v, [pairs_c[p, 1, pl.ds(i * 16, 16)]], ones)
        bn = (b + 2) % NB
        pn = (p + 2) % NP
        if guard_drain:
            @pl.when(j >= 1)
            def _drain():
                scatter_desc(bn, (p + 5) % NP).wait()
        else:
            scatter_desc(bn, (p + 5) % NP).wait()
        if start2:
            pairs_desc(j + 2, pn).wait()
            gather_desc(bn, pn).start()
        if load4:
            pairs_desc(j + 4, (p + 4) % NP).start()

    for j in range(4):
        pairs_desc(j, j).start()
    for j in range(2):
        pairs_desc(j, j).wait()
        gather_desc(j, j).start()

    def six(i, carry):
        j0 = i * UNROLL
        for u in range(UNROLL):
            step(j0 + u, u % NB, u, guard_drain=(u == 0))
        return carry

    lax.fori_loop(0, NLOOP, six, 0)
    for j in range(NLOOP * UNROLL, NCH):
        step(j, j % NB, j % NP, start2=(j + 2 < NCH), load4=(j + 4 < NCH))
    scatter_desc((NCH - 1) % NB, (NCH - 1) % NP).wait()

    plsc.subcore_barrier()

    @pl.when(s < NZ)
    def _writeout():
        row0 = pl.multiple_of(s * ZR, 8)
        pltpu.sync_copy(agg_sh.at[pl.ds(row0, ZR)],
                        agg_out.at[c, pl.ds(row0, ZR)])

    pltpu.sync_copy(deg_v, deg_out.at[w, 0])


@jax.jit
def _sc_agg(x, pairs, zrow, zdeg):
    mesh = plsc.VectorSubcoreMesh(core_axis_name="c", subcore_axis_name="s")
    f = pl.kernel(
        _sc_agg_body,
        out_type=(
            jax.ShapeDtypeStruct((NC, N, D), jnp.float32),
            jax.ShapeDtypeStruct((NW, 1, N), jnp.float32),
        ),
        mesh=mesh,
        compiler_params=pltpu.CompilerParams(needs_layout_passes=False),
        scratch_types=(
            [pltpu.VMEM((NP, 2, CH), jnp.int32),
             pltpu.VMEM((NB, CH, D), jnp.float32),
             pltpu.VMEM((N,), jnp.float32),
             pltpu.VMEM_SHARED((N, D), jnp.float32)]
            + [pltpu.SemaphoreType.DMA] * (2 * NB + NP)),
    )
    return f(x, pairs, zrow, zdeg)


BLK = 2048


def _tc_mlp_body(x_ref, aggp_ref, degp_ref, ws_ref, wn_ref, b1_ref,
                 wc1_ref, bc1_ref, a_ref, wc2_ref, bc2_ref, out_ref):
    hi = jax.lax.Precision.HIGHEST
    x = x_ref[...]
    agg = aggp_ref[0] + aggp_ref[1]
    deg = jnp.sum(degp_ref[...], axis=0)
    agg = agg / jnp.maximum(deg, 1.0)[:, None]
    h1 = jax.nn.relu(
        jnp.dot(x, ws_ref[...], precision=jax.lax.Precision.DEFAULT,
                preferred_element_type=jnp.float32) + b1_ref[0, :D])
    h2 = jax.nn.relu(
        jnp.dot(agg, wn_ref[...], precision=jax.lax.Precision.DEFAULT,
                preferred_element_type=jnp.float32) + b1_ref[0, D:])
    z = (jnp.dot(h1, wc1_ref[pl.ds(0, D)], precision=jax.lax.Precision.DEFAULT,
                 preferred_element_type=jnp.float32)
         + jnp.dot(h2, wc1_ref[pl.ds(D, D)], precision=jax.lax.Precision.DEFAULT,
                   preferred_element_type=jnp.float32)
         + bc1_ref[0])
    z = jnp.where(z >= 0, z, a_ref[0] * z)
    out_ref[...] = (jnp.dot(z, wc2_ref[...], precision=jax.lax.Precision.DEFAULT,
                            preferred_element_type=jnp.float32) + bc2_ref[0])


@jax.jit
def _tc_mlp(x, agg_p, deg_p, W_self, W_neigh, b1, Wc1, bc1, prelu_a, Wc2, bc2):
    grid = ((N + BLK - 1) // BLK,)
    return pl.pallas_call(
        _tc_mlp_body,
        grid=grid,
        in_specs=[
            pl.BlockSpec((BLK, D), lambda i: (i, 0)),
            pl.BlockSpec((NC, BLK, D), lambda i: (0, i, 0)),
            pl.BlockSpec((NW, BLK), lambda i: (0, i)),
            pl.BlockSpec((D, D), lambda i: (0, 0)),
            pl.BlockSpec((D, D), lambda i: (0, 0)),
            pl.BlockSpec((1, 2 * D), lambda i: (0, 0)),
            pl.BlockSpec((2 * D, D), lambda i: (0, 0)),
            pl.BlockSpec((1, D), lambda i: (0, 0)),
            pl.BlockSpec((1, D), lambda i: (0, 0)),
            pl.BlockSpec((D, 2), lambda i: (0, 0)),
            pl.BlockSpec((1, 2), lambda i: (0, 0)),
        ],
        out_specs=pl.BlockSpec((BLK, 2), lambda i: (i, 0)),
        out_shape=jax.ShapeDtypeStruct((N, 2), jnp.float32),
    )(x, agg_p, deg_p, W_self, W_neigh, b1, Wc1, bc1, prelu_a, Wc2, bc2)


def kernel(x, edge_index, W_self, W_neigh, b1, Wc1, bc1, prelu_a, Wc2, bc2):
    pairs = jnp.stack(
        [edge_index[0].reshape(NW, NCH, CH),
         edge_index[1].reshape(NW, NCH, CH)], axis=2)
    zrow = jnp.zeros((ZR, D), jnp.float32)
    zdeg = jnp.zeros((N,), jnp.float32)
    agg_p, deg_3d = _sc_agg(x, pairs, zrow, zdeg)
    deg_p = deg_3d.reshape(NW, N)
    out = _tc_mlp(x, agg_p, deg_p, W_self, W_neigh,
                  b1.reshape(1, 2 * D), Wc1, bc1.reshape(1, D),
                  prelu_a.reshape(1, D), Wc2, bc2.reshape(1, 2))
    return out.reshape(-1)
